# baseline (speedup 1.0000x reference)
import functools

import jax
import jax.numpy as jnp
import numpy as np
from jax import lax
from jax.experimental import pallas as pl
from jax.experimental.pallas import tpu as pltpu
from jax.experimental.pallas import tpu_sc as plsc

T = 200
NUM_GRAPHS = 512
D_FEAT = 128
HIDDEN = 64

_INTERPRET = False


def _sched_table():
    tt = np.arange(T + 1, dtype=np.float64)
    alpha_bar = (1.0 - (tt / T) ** 2.0) ** 2
    alpha_bar = np.clip(alpha_bar, 1e-4, 1.0)
    alpha = np.clip(alpha_bar[1:] / alpha_bar[:-1], 1e-3, 1.0)
    alpha_bar = np.cumprod(alpha)
    out = np.zeros((256, 2), np.float32)
    out[:T, 0] = np.sqrt(alpha_bar)
    out[:T, 1] = np.sqrt(1.0 - alpha_bar)
    return out


_SCHED_NP = _sched_table()


def _dot(a, b):
    return jnp.dot(a, b, preferred_element_type=jnp.float32)


_NSC = 2
_NSS = 16
_NW = _NSC * _NSS
_CHUNK = 3136
_SROWS = 520


_NCOL = 6


def _sc_segsum(p6f, seg, n):
    tail = n - (_NW - 1) * _CHUNK
    ng_full = _CHUNK // 16
    ng_tail = tail // 16

    @functools.partial(
        pl.kernel,
        out_type=jax.ShapeDtypeStruct((_NW, _SROWS * 8), jnp.float32),
        mesh=plsc.VectorSubcoreMesh(core_axis_name="c", subcore_axis_name="s"),
        compiler_params=pltpu.CompilerParams(needs_layout_passes=False),
        scratch_types=[pltpu.VMEM((_CHUNK,), jnp.float32)] * _NCOL
        + [pltpu.VMEM((_CHUNK,), jnp.int32),
           pltpu.VMEM((_SROWS * 8,), jnp.float32),
           pltpu.SemaphoreType.DMA],
    )
    def k(p6f_hbm, seg_hbm, out_hbm, *refs):
        bufs = refs[:_NCOL]
        idxf = refs[_NCOL]
        tbl = refs[_NCOL + 1]
        sem = refs[_NCOL + 2]
        cid = lax.axis_index("c")
        sid = lax.axis_index("s")
        wid = cid * _NSS + sid

        z = jnp.zeros((16,), jnp.float32)

        def zero(j, carry):
            tbl[pl.ds(j * 16, 16)] = z
            return carry

        def stage(base, rows):
            hs = [pltpu.async_copy(p6f_hbm.at[pl.ds(c * n + base, rows)],
                                   bufs[c].at[pl.ds(0, rows)], sem)
                  for c in range(_NCOL)]
            hs.append(pltpu.async_copy(seg_hbm.at[pl.ds(base, rows)],
                                       idxf.at[pl.ds(0, rows)], sem))
            lax.fori_loop(0, (_SROWS * 8) // 16, zero, 0)
            for h in hs:
                h.wait()

        @pl.when(wid < _NW - 1)
        def _stage_full():
            stage(wid * _CHUNK, _CHUNK)

        @pl.when(wid == _NW - 1)
        def _stage_tail():
            stage((_NW - 1) * _CHUNK, tail)

        ones = jnp.full((16,), 1.0, jnp.float32)

        def group(g, carry):
            seg16 = idxf[pl.ds(g * 16, 16)]
            base = seg16 * 8
            for c in range(_NCOL):
                v = bufs[c][pl.ds(g * 16, 16)]
                plsc.addupdate_scatter(tbl, [base + c], v)
            plsc.addupdate_scatter(tbl, [base + 6], ones)
            return carry

        @pl.when(wid < _NW - 1)
        def _run_full():
            lax.fori_loop(0, ng_full, group, 0)

        @pl.when(wid == _NW - 1)
        def _run_tail():
            lax.fori_loop(0, ng_tail, group, 0)

        pltpu.sync_copy(tbl, out_hbm.at[wid])

    return k(p6f, seg)


def _main_body(parts_ref, t_ref, sched_ref, w1a_ref, w1b_ref, b1_ref,
               w2x_ref, b2x_ref, w2p_ref, b2p_ref,
               lx_ref, xe_ref, lpos_ref, leps_ref, segf_ref,
               ox_ref, op_ref, tbl_ref):
    i = pl.program_id(0)

    @pl.when(i == 0)
    def _finalize_table():
        sums = jnp.sum(parts_ref[...], axis=0)
        cnt = jnp.maximum(sums[:, 6:7], 1.0)
        means = sums[:, 0:6] / cnt
        t_f = t_ref[...].astype(jnp.float32)
        iota_t = jax.lax.broadcasted_iota(
            jnp.int32, (1, 256), 1).astype(jnp.float32)
        onehot_t = (t_f == iota_t).astype(jnp.float32)
        sch = _dot(onehot_t, sched_ref[...])
        tfeat = t_f * (1.0 / T)
        pad = jnp.zeros((_SROWS, 7), jnp.float32)
        tbl_ref[...] = jnp.concatenate([means, sch, tfeat, pad], axis=1)

    seg_col = segf_ref[...]
    iota_g = jax.lax.broadcasted_iota(
        jnp.int32, (1, _SROWS), 1).astype(jnp.float32)
    onehot = (seg_col == iota_g).astype(jnp.float32)
    vals = _dot(onehot, tbl_ref[...])

    mean_pos = vals[:, 0:3]
    mean_eps = vals[:, 3:6]
    sab = vals[:, 6:7]
    somab = vals[:, 7:8]
    tfeat = vals[:, 8:9]

    pos = lpos_ref[...]
    eps = leps_ref[...]
    pos_eps = eps - mean_eps
    x_t_pos = sab * (pos - mean_pos) + somab * pos_eps
    xtp4 = jnp.concatenate([x_t_pos, tfeat], axis=1)

    xe = xe_ref[...]
    x_t_x = sab * lx_ref[...] + somab * xe

    pre = _dot(x_t_x, w1a_ref[...]) + _dot(xtp4, w1b_ref[...]) + b1_ref[...]
    h = jnp.maximum(pre, 0.0)

    xp = _dot(h, w2x_ref[...]) + b2x_ref[...]
    pp = _dot(h, w2p_ref[...]) + b2p_ref[...]

    ex = jnp.sum((xe - xp) ** 2)
    ep = jnp.sum((pos_eps - pp) ** 2)

    @pl.when(i == 0)
    def _init():
        ox_ref[...] = ex.reshape(1, 1)
        op_ref[...] = ep.reshape(1, 1)

    @pl.when(i != 0)
    def _acc():
        ox_ref[...] = ox_ref[...] + ex.reshape(1, 1)
        op_ref[...] = op_ref[...] + ep.reshape(1, 1)


def _main_pass(parts, tpad2, sched, lx, xe, lpos, leps, segf,
               w1a, w1b4, b1, w2x, b2x, w2p, b2p, block):
    n = lx.shape[0]
    nb = n // block
    full = lambda *s: pl.BlockSpec(s, lambda i: (0,) * len(s))
    return pl.pallas_call(
        _main_body,
        grid=(nb,),
        in_specs=[
            full(_NW, _SROWS, 8),
            full(_SROWS, 1),
            full(256, 2),
            full(D_FEAT, HIDDEN),
            full(4, HIDDEN),
            full(1, HIDDEN),
            full(HIDDEN, D_FEAT),
            full(1, D_FEAT),
            full(HIDDEN, 3),
            full(1, 3),
            pl.BlockSpec((block, D_FEAT), lambda i: (i, 0)),
            pl.BlockSpec((block, D_FEAT), lambda i: (i, 0)),
            pl.BlockSpec((block, 3), lambda i: (i, 0)),
            pl.BlockSpec((block, 3), lambda i: (i, 0)),
            pl.BlockSpec((block, 1), lambda i: (i, 0)),
        ],
        out_specs=[
            pl.BlockSpec((1, 1), lambda i: (0, 0)),
            pl.BlockSpec((1, 1), lambda i: (0, 0)),
        ],
        out_shape=[
            jax.ShapeDtypeStruct((1, 1), jnp.float32),
            jax.ShapeDtypeStruct((1, 1), jnp.float32),
        ],
        scratch_shapes=[pltpu.VMEM((_SROWS, 16), jnp.float32)],
        compiler_params=pltpu.CompilerParams(
            dimension_semantics=("arbitrary",)),
        interpret=_INTERPRET,
    )(parts, tpad2, sched, w1a, w1b4, b1, w2x, b2x, w2p, b2p,
      lx, xe, lpos, leps, segf)


BLOCK = 5000


def kernel(ligand_x, ligand_pos, protein_x, protein_pos, x_eps, pos_eps_raw,
           W1, b1, W2x, b2x, W2pos, b2pos, ligand_batch, protein_batch, t):
    n = ligand_x.shape[0]
    seg_i = ligand_batch.astype(jnp.int32)
    segf = seg_i.astype(jnp.float32)[:, None]
    p6f = jnp.concatenate([ligand_pos.T, pos_eps_raw.T]).reshape(-1)
    tpad = jnp.concatenate(
        [t[:, 0].astype(jnp.int32),
         jnp.zeros((_SROWS - NUM_GRAPHS,), jnp.int32)])
    sched = jnp.asarray(_SCHED_NP)

    parts = _sc_segsum(p6f, seg_i, n)

    w1a = W1[0:D_FEAT]
    w1b4 = W1[D_FEAT:D_FEAT + 4]
    ox, op = _main_pass(parts.reshape(_NW, _SROWS, 8), tpad[:, None], sched,
                        ligand_x, x_eps, ligand_pos, pos_eps_raw, segf,
                        w1a, w1b4, b1[None, :], W2x, b2x[None, :],
                        W2pos, b2pos[None, :], BLOCK)

    sum_x = ox[0, 0]
    sum_pos = op[0, 0]
    L_x = sum_x / (n * D_FEAT)
    L_pos = sum_pos / (n * 3)
    L_simple = 0.25 * (L_pos + L_x)
    L_unweighted = 0.5 * (sum_x + sum_pos) / (n * (D_FEAT + 3))
    return (L_simple, L_unweighted, L_pos, L_x)


# speedup vs baseline: 1.2860x; 1.2860x over previous
import functools

import jax
import jax.numpy as jnp
import numpy as np
from jax import lax
from jax.experimental import pallas as pl
from jax.experimental.pallas import tpu as pltpu
from jax.experimental.pallas import tpu_sc as plsc

T = 200
NUM_GRAPHS = 512
D_FEAT = 128
HIDDEN = 64

_INTERPRET = False


def _sched_table():
    tt = np.arange(T + 1, dtype=np.float64)
    alpha_bar = (1.0 - (tt / T) ** 2.0) ** 2
    alpha_bar = np.clip(alpha_bar, 1e-4, 1.0)
    alpha = np.clip(alpha_bar[1:] / alpha_bar[:-1], 1e-3, 1.0)
    alpha_bar = np.cumprod(alpha)
    out = np.zeros((256, 2), np.float32)
    out[:T, 0] = np.sqrt(alpha_bar)
    out[:T, 1] = np.sqrt(1.0 - alpha_bar)
    return out


_SCHED_NP = _sched_table()


def _dot(a, b):
    return jnp.dot(a, b, preferred_element_type=jnp.float32)


_NSC = 2
_NSS = 16
_NW = _NSC * _NSS
_CHUNK = 3136
_SROWS = 520


_NCOL = 6


def _sc_segsum(p6f, seg, n):
    tail = n - (_NW - 1) * _CHUNK
    ng_full = _CHUNK // 16
    ng_tail = tail // 16

    @functools.partial(
        pl.kernel,
        out_type=jax.ShapeDtypeStruct((_NW, _SROWS * 8), jnp.float32),
        mesh=plsc.VectorSubcoreMesh(core_axis_name="c", subcore_axis_name="s"),
        compiler_params=pltpu.CompilerParams(needs_layout_passes=False),
        scratch_types=[pltpu.VMEM((_CHUNK,), jnp.float32)] * _NCOL
        + [pltpu.VMEM((_CHUNK,), jnp.int32),
           pltpu.VMEM((_SROWS * 8,), jnp.float32),
           pltpu.SemaphoreType.DMA],
    )
    def k(p6f_hbm, seg_hbm, out_hbm, *refs):
        bufs = refs[:_NCOL]
        idxf = refs[_NCOL]
        tbl = refs[_NCOL + 1]
        sem = refs[_NCOL + 2]
        cid = lax.axis_index("c")
        sid = lax.axis_index("s")
        wid = cid * _NSS + sid

        z = jnp.zeros((16,), jnp.float32)

        def zero(j, carry):
            tbl[pl.ds(j * 16, 16)] = z
            return carry

        def stage(base, rows):
            hs = [pltpu.async_copy(p6f_hbm.at[pl.ds(c * n + base, rows)],
                                   bufs[c].at[pl.ds(0, rows)], sem)
                  for c in range(_NCOL)]
            hs.append(pltpu.async_copy(seg_hbm.at[pl.ds(base, rows)],
                                       idxf.at[pl.ds(0, rows)], sem))
            lax.fori_loop(0, (_SROWS * 8) // 16, zero, 0)
            for h in hs:
                h.wait()

        @pl.when(wid < _NW - 1)
        def _stage_full():
            stage(wid * _CHUNK, _CHUNK)

        @pl.when(wid == _NW - 1)
        def _stage_tail():
            stage((_NW - 1) * _CHUNK, tail)

        ones = jnp.full((16,), 1.0, jnp.float32)

        def group(g, carry):
            seg16 = idxf[pl.ds(g * 16, 16)]
            base = seg16 * 8
            for c in range(_NCOL):
                v = bufs[c][pl.ds(g * 16, 16)]
                plsc.addupdate_scatter(tbl, [base + c], v)
            plsc.addupdate_scatter(tbl, [base + 6], ones)
            return carry

        @pl.when(wid < _NW - 1)
        def _run_full():
            lax.fori_loop(0, ng_full, group, 0)

        @pl.when(wid == _NW - 1)
        def _run_tail():
            lax.fori_loop(0, ng_tail, group, 0)

        pltpu.sync_copy(tbl, out_hbm.at[wid])

    return k(p6f, seg)


def _main_body(parts_ref, t_ref, sched_ref, w1a_ref, w1b_ref, b1_ref,
               w2x_ref, b2x_ref, w2p_ref, b2p_ref,
               lx_ref, xe_ref, p8_ref,
               ox_ref, op_ref, tbl_ref):
    i = pl.program_id(0)

    @pl.when(i == 0)
    def _finalize_table():
        sums = jnp.sum(parts_ref[...], axis=0)
        cnt = jnp.maximum(sums[:, 6:7], 1.0)
        means = sums[:, 0:6] / cnt
        t_f = t_ref[...].astype(jnp.float32)
        iota_t = jax.lax.broadcasted_iota(
            jnp.int32, (1, 256), 1).astype(jnp.float32)
        onehot_t = (t_f == iota_t).astype(jnp.float32)
        sch = _dot(onehot_t, sched_ref[...])
        tfeat = t_f * (1.0 / T)
        pad = jnp.zeros((_SROWS, 7), jnp.float32)
        tbl_ref[...] = jnp.concatenate([means, sch, tfeat, pad], axis=1)

    p8 = p8_ref[...]
    seg_col = p8[:, 7:8]
    iota_g = jax.lax.broadcasted_iota(
        jnp.int32, (1, _SROWS), 1).astype(jnp.float32)
    onehot = (seg_col == iota_g).astype(jnp.float32)
    vals = _dot(onehot, tbl_ref[...])

    mean_pos = vals[:, 0:3]
    mean_eps = vals[:, 3:6]
    sab = vals[:, 6:7]
    somab = vals[:, 7:8]
    tfeat = vals[:, 8:9]

    pos = p8[:, 0:3]
    eps = p8[:, 3:6]
    pos_eps = eps - mean_eps
    x_t_pos = sab * (pos - mean_pos) + somab * pos_eps
    xtp4 = jnp.concatenate([x_t_pos, tfeat], axis=1)

    xe = xe_ref[...]
    x_t_x = sab * lx_ref[...] + somab * xe

    pre = _dot(x_t_x, w1a_ref[...]) + _dot(xtp4, w1b_ref[...]) + b1_ref[...]
    h = jnp.maximum(pre, 0.0)

    xp = _dot(h, w2x_ref[...]) + b2x_ref[...]
    pp = _dot(h, w2p_ref[...]) + b2p_ref[...]

    ex = jnp.sum((xe - xp) ** 2)
    ep = jnp.sum((pos_eps - pp) ** 2)

    @pl.when(i == 0)
    def _init():
        ox_ref[...] = ex.reshape(1, 1)
        op_ref[...] = ep.reshape(1, 1)

    @pl.when(i != 0)
    def _acc():
        ox_ref[...] = ox_ref[...] + ex.reshape(1, 1)
        op_ref[...] = op_ref[...] + ep.reshape(1, 1)


def _main_pass(parts, tpad2, sched, lx, xe, p8,
               w1a, w1b4, b1, w2x, b2x, w2p, b2p, block):
    n = lx.shape[0]
    nb = n // block
    full = lambda *s: pl.BlockSpec(s, lambda i: (0,) * len(s))
    return pl.pallas_call(
        _main_body,
        grid=(nb,),
        in_specs=[
            full(_NW, _SROWS, 8),
            full(_SROWS, 1),
            full(256, 2),
            full(D_FEAT, HIDDEN),
            full(4, HIDDEN),
            full(1, HIDDEN),
            full(HIDDEN, D_FEAT),
            full(1, D_FEAT),
            full(HIDDEN, 3),
            full(1, 3),
            pl.BlockSpec((block, D_FEAT), lambda i: (i, 0)),
            pl.BlockSpec((block, D_FEAT), lambda i: (i, 0)),
            pl.BlockSpec((block, 8), lambda i: (i, 0)),
        ],
        out_specs=[
            pl.BlockSpec((1, 1), lambda i: (0, 0)),
            pl.BlockSpec((1, 1), lambda i: (0, 0)),
        ],
        out_shape=[
            jax.ShapeDtypeStruct((1, 1), jnp.float32),
            jax.ShapeDtypeStruct((1, 1), jnp.float32),
        ],
        scratch_shapes=[pltpu.VMEM((_SROWS, 16), jnp.float32)],
        compiler_params=pltpu.CompilerParams(
            dimension_semantics=("arbitrary",)),
        interpret=_INTERPRET,
    )(parts, tpad2, sched, w1a, w1b4, b1, w2x, b2x, w2p, b2p,
      lx, xe, p8)


BLOCK = 10000


def kernel(ligand_x, ligand_pos, protein_x, protein_pos, x_eps, pos_eps_raw,
           W1, b1, W2x, b2x, W2pos, b2pos, ligand_batch, protein_batch, t):
    n = ligand_x.shape[0]
    seg_i = ligand_batch.astype(jnp.int32)
    segf = seg_i.astype(jnp.float32)[:, None]
    ones = jnp.ones((n, 1), jnp.float32)
    p8 = jnp.concatenate([ligand_pos, pos_eps_raw, ones, segf], axis=1)
    p6f = p8[:, :_NCOL].T.reshape(-1)
    tpad = jnp.concatenate(
        [t[:, 0].astype(jnp.int32),
         jnp.zeros((_SROWS - NUM_GRAPHS,), jnp.int32)])
    sched = jnp.asarray(_SCHED_NP)

    parts = _sc_segsum(p6f, seg_i, n)

    w1a = W1[0:D_FEAT]
    w1b4 = W1[D_FEAT:D_FEAT + 4]
    ox, op = _main_pass(parts.reshape(_NW, _SROWS, 8), tpad[:, None], sched,
                        ligand_x, x_eps, p8,
                        w1a, w1b4, b1[None, :], W2x, b2x[None, :],
                        W2pos, b2pos[None, :], BLOCK)

    sum_x = ox[0, 0]
    sum_pos = op[0, 0]
    L_x = sum_x / (n * D_FEAT)
    L_pos = sum_pos / (n * 3)
    L_simple = 0.25 * (L_pos + L_x)
    L_unweighted = 0.5 * (sum_x + sum_pos) / (n * (D_FEAT + 3))
    return (L_simple, L_unweighted, L_pos, L_x)
